# Initial kernel scaffold; baseline (speedup 1.0000x reference)
import jax
import jax.numpy as jnp
from jax.experimental import pallas as pl


def kernel(output, target):
    raise NotImplementedError("write your pallas kernel here")


# speedup vs baseline: 2.1551x; 2.1551x over previous
import math

import jax
import jax.numpy as jnp
from jax.experimental import pallas as pl
from jax.experimental.pallas import tpu as pltpu

_BETA = 0.99
_C = 1000
_B = 16384
_ROWS = 1024
_GRID = _B // _ROWS
_LN_BETA = math.log(_BETA)


def _body(x_ref, t_ref, loss_ref, cnt_ref, s_ref):
    i = pl.program_id(0)

    @pl.when(i == 0)
    def _init():
        cnt_ref[...] = jnp.zeros_like(cnt_ref)
        s_ref[...] = jnp.zeros_like(s_ref)

    x = x_ref[...]
    t = t_ref[0, 0, :]
    m = jnp.max(x, axis=1, keepdims=True)
    e = jnp.exp(x - m)
    lse = m[:, 0] + jnp.log(jnp.sum(e, axis=1))
    cols = jax.lax.broadcasted_iota(jnp.int32, (_ROWS, _C), 1)
    mask = cols == t[:, None]
    maskf = mask.astype(jnp.float32)
    picked = jnp.sum(jnp.where(mask, x, 0.0), axis=1)
    nll = lse - picked
    cnt_ref[...] += jnp.sum(maskf, axis=0, keepdims=True)
    s_ref[...] += jnp.sum(nll[:, None] * maskf, axis=0, keepdims=True)

    @pl.when(i == _GRID - 1)
    def _fin():
        cnt = cnt_ref[...]
        s = s_ref[...]
        freq = cnt * (1.0 / _B)
        eff = 1.0 - jnp.exp(freq * _LN_BETA)
        valid = cnt > 0.0
        w = jnp.where(valid, (1.0 - _BETA) / eff, 0.0)
        num = jnp.sum(w * s)
        den = jnp.sum(w * cnt)
        loss_ref[...] = (num / den)[None, None]


def kernel(output, target):
    t3 = target.astype(jnp.int32).reshape(_GRID, 1, _ROWS)
    loss = pl.pallas_call(
        _body,
        grid=(_GRID,),
        in_specs=[
            pl.BlockSpec((_ROWS, _C), lambda i: (i, 0)),
            pl.BlockSpec((1, 1, _ROWS), lambda i: (i, 0, 0)),
        ],
        out_specs=pl.BlockSpec((1, 1), lambda i: (0, 0)),
        out_shape=jax.ShapeDtypeStruct((1, 1), jnp.float32),
        scratch_shapes=[
            pltpu.VMEM((1, _C), jnp.float32),
            pltpu.VMEM((1, _C), jnp.float32),
        ],
    )(output, t3)
    return loss[0, 0]
